# baseline (speedup 1.0000x reference)
import jax
import jax.numpy as jnp
from jax.experimental import pallas as pl
from jax.experimental.pallas import tpu as pltpu

NUM_EXPERTS = 16
TOPK = 2
BM = 1024


def _router_block(x_ref, w_ref, b_ref, logits_ref, mask_ref):
    xb = x_ref[...]
    w = w_ref[...]
    logits = jax.lax.dot_general(
        xb, w, (((1,), (1,)), ((), ())),
        preferred_element_type=jnp.float32)
    logits = logits + b_ref[...]
    logits_ref[...] = logits
    mask_ref[...] = jnp.zeros(mask_ref.shape, jnp.int32)


@jax.jit
def kernel(x, W, b):
    M, K = x.shape
    E = W.shape[0]
    grid = (M // BM,)
    logits, mask = pl.pallas_call(
        _router_block,
        grid=grid,
        in_specs=[
            pl.BlockSpec((BM, K), lambda i: (i, 0)),
            pl.BlockSpec((E, K), lambda i: (0, 0)),
            pl.BlockSpec((1, E), lambda i: (0, 0)),
        ],
        out_specs=[
            pl.BlockSpec((BM, E), lambda i: (i, 0)),
            pl.BlockSpec((BM, TOPK * E), lambda i: (i, 0)),
        ],
        out_shape=[
            jax.ShapeDtypeStruct((M, E), jnp.float32),
            jax.ShapeDtypeStruct((M, TOPK * E), jnp.int32),
        ],
        compiler_params=pltpu.CompilerParams(
            dimension_semantics=("parallel",),
        ),
    )(x, W, b.reshape(1, E))
    return (logits, mask)


# speedup vs baseline: 1.2474x; 1.2474x over previous
import jax
import jax.numpy as jnp
from jax.experimental import pallas as pl
from jax.experimental.pallas import tpu as pltpu

NUM_EXPERTS = 16
TOPK = 2
BM = 1024


def _router_block(x_ref, w_ref, b_ref, logits_ref, wts_ref, idx_ref, mask_ref):
    xb = x_ref[...]
    w = w_ref[...]
    lg = jax.lax.dot_general(
        xb, w, (((1,), (1,)), ((), ())),
        preferred_element_type=jnp.float32)
    lt = lg.T + b_ref[...]
    logits_ref[...] = lt

    e_iota = jax.lax.broadcasted_iota(jnp.int32, lt.shape, 0)
    big = jnp.int32(NUM_EXPERTS)
    m1 = jnp.max(lt, axis=0, keepdims=True)
    i1 = jnp.min(jnp.where(lt == m1, e_iota, big), axis=0, keepdims=True)
    masked = jnp.where(e_iota == i1, -jnp.inf, lt)
    m2 = jnp.max(masked, axis=0, keepdims=True)
    i2 = jnp.min(jnp.where(masked == m2, e_iota, big), axis=0, keepdims=True)

    r = jnp.exp(m2 - m1)
    denom = 1.0 + r
    wts_ref[...] = jnp.concatenate([1.0 / denom, r / denom], axis=0)
    idx_ref[...] = jnp.concatenate([i1, i2], axis=0)

    mask_ref[0] = (e_iota == i1).astype(jnp.int32)
    mask_ref[1] = (e_iota == i2).astype(jnp.int32)


@jax.jit
def kernel(x, W, b):
    M, K = x.shape
    E = W.shape[0]
    grid = (M // BM,)
    lt, wts_t, idx_t, mask_t = pl.pallas_call(
        _router_block,
        grid=grid,
        in_specs=[
            pl.BlockSpec((BM, K), lambda i: (i, 0)),
            pl.BlockSpec((E, K), lambda i: (0, 0)),
            pl.BlockSpec((E, 1), lambda i: (0, 0)),
        ],
        out_specs=[
            pl.BlockSpec((E, BM), lambda i: (0, i)),
            pl.BlockSpec((TOPK, BM), lambda i: (0, i)),
            pl.BlockSpec((TOPK, BM), lambda i: (0, i)),
            pl.BlockSpec((TOPK, E, BM), lambda i: (0, 0, i)),
        ],
        out_shape=[
            jax.ShapeDtypeStruct((E, M), jnp.float32),
            jax.ShapeDtypeStruct((TOPK, M), jnp.float32),
            jax.ShapeDtypeStruct((TOPK, M), jnp.int32),
            jax.ShapeDtypeStruct((TOPK, E, M), jnp.int32),
        ],
        compiler_params=pltpu.CompilerParams(
            dimension_semantics=("parallel",),
        ),
    )(x, W, b.reshape(E, 1))
    return (lt.T, wts_t.T, idx_t.T, jnp.transpose(mask_t, (2, 0, 1)))
